# baseline (speedup 1.0000x reference)
import functools

import jax
import jax.numpy as jnp
from jax.experimental import pallas as pl
from jax.experimental.pallas import tpu as pltpu

_BB = 4


def _se_block_kernel(x_ref, w1_ref, w2_ref, o_ref, *, inv_hw, bb):
    mean = jnp.sum(x_ref[...], axis=-1, keepdims=True,
                   dtype=jnp.float32) * inv_hw
    w1 = w1_ref[...]
    w2 = w2_ref[...]
    for i in range(bb):
        h = jax.lax.dot_general(w1, mean[i], (((1,), (0,)), ((), ())),
                                preferred_element_type=jnp.float32)
        h = h * jax.nn.sigmoid(h)
        s = jax.lax.dot_general(w2, h, (((1,), (0,)), ((), ())),
                                preferred_element_type=jnp.float32)
        gate = jax.nn.sigmoid(s)
        o_ref[i] = x_ref[i] * gate.astype(o_ref.dtype)


def kernel(x_nchw, w1, w2):
    B, C, H, W = x_nchw.shape
    HW = H * W
    hidden = w1.shape[0]
    dtype = x_nchw.dtype
    inv_hw = float(1.0 / HW)
    bb = _BB if B % _BB == 0 else 1

    x_flat = x_nchw.reshape(B, C, HW)

    out_flat = pl.pallas_call(
        functools.partial(_se_block_kernel, inv_hw=inv_hw, bb=bb),
        out_shape=jax.ShapeDtypeStruct((B, C, HW), dtype),
        grid=(B // bb,),
        in_specs=[
            pl.BlockSpec((bb, C, HW), lambda b: (b, 0, 0)),
            pl.BlockSpec((hidden, C), lambda b: (0, 0)),
            pl.BlockSpec((C, hidden), lambda b: (0, 0)),
        ],
        out_specs=pl.BlockSpec((bb, C, HW), lambda b: (b, 0, 0)),
        compiler_params=pltpu.CompilerParams(
            dimension_semantics=("parallel",),
            vmem_limit_bytes=58 << 20,
        ),
    )(x_flat, w1, w2)

    return out_flat.reshape(B, C, H, W)
